# Initial kernel scaffold; baseline (speedup 1.0000x reference)
import jax
import jax.numpy as jnp
from jax.experimental import pallas as pl


def kernel(data_x1, mask_x1, data_x2, mask_x2, word_embed, W1, b1, W2, b2):
    raise NotImplementedError("write your pallas kernel here")


# speedup vs baseline: 1.5230x; 1.5230x over previous
import functools

import jax
import jax.numpy as jnp
from jax import lax
from jax.experimental import pallas as pl
from jax.experimental.pallas import tpu as pltpu
from jax.experimental.pallas import tpu_sc as plsc

B = 4096
L = 50
D = 64
F = 2
NC = 2
NS = 16
NB = B // NS
CT = 2
CR = CT * L
NCH = NB // CT
CCH = D // 16


def _pool_body(idx_hbm, mask_hbm, table_hbm, out_hbm,
               idx_v, mask_v, rows0, rows1, outb, sem0, sem1):
    f = lax.axis_index("c")
    g = lax.axis_index("s")
    pltpu.sync_copy(idx_hbm.at[f, g], idx_v)
    pltpu.sync_copy(mask_hbm.at[f, g], mask_v)

    bufs = ((rows0, sem0), (rows1, sem1))

    def _issue(t, rows, sem):
        return pltpu.async_copy(table_hbm.at[idx_v.at[t]], rows, sem)

    _issue(0, rows0, sem0)
    _issue(1, rows1, sem1)

    def _accum(t, rows):
        for j in range(CT):
            tl = t * CT + j
            mrow = [mask_v[tl, pl.ds(k * 16, 16)] for k in range(4)]
            accs = [jnp.zeros((16,), jnp.float32) for _ in range(CCH)]
            for r in range(L):
                m = mrow[r // 16][r % 16]
                for c in range(CCH):
                    accs[c] = accs[c] + rows[j * L + r, pl.ds(c * 16, 16)] * m
            for c in range(CCH):
                outb[tl, pl.ds(c * 16, 16)] = accs[c]

    def _step(t2, carry):
        for p, (rows, sem) in enumerate(bufs):
            t = t2 * 2 + p
            pltpu.make_async_copy(table_hbm.at[idx_v.at[t]], rows, sem).wait()
            _accum(t, rows)

            @pl.when(t2 < NCH // 2 - 1)
            def _():
                _issue(t + 2, rows, sem)
        return carry

    lax.fori_loop(0, NCH // 2, _step, 0)
    pltpu.sync_copy(outb, out_hbm.at[f, pl.ds(g * NB, NB), :])


_pool = functools.partial(
    pl.kernel,
    out_type=jax.ShapeDtypeStruct((F, B, D), jnp.float32),
    mesh=plsc.VectorSubcoreMesh(core_axis_name="c", subcore_axis_name="s"),
    scratch_types=[
        pltpu.VMEM((NCH, CR), jnp.int32),
        pltpu.VMEM((NB, D), jnp.float32),
        pltpu.VMEM((CR, D), jnp.float32),
        pltpu.VMEM((CR, D), jnp.float32),
        pltpu.VMEM((NB, D), jnp.float32),
        pltpu.SemaphoreType.DMA,
        pltpu.SemaphoreType.DMA,
    ],
    compiler_params=pltpu.CompilerParams(use_tc_tiling_on_sc=False),
)(_pool_body)


def _mlp_body(p1_ref, p2_ref, w1_ref, b1_ref, w2_ref, b2_ref, out_ref):
    w1 = w1_ref[...]
    h = jnp.dot(p1_ref[...], w1[:D], preferred_element_type=jnp.float32)
    h = h + jnp.dot(p2_ref[...], w1[D:], preferred_element_type=jnp.float32)
    h = h + b1_ref[...]
    o = jnp.dot(h, w2_ref[...], preferred_element_type=jnp.float32)
    out_ref[...] = o + b2_ref[...]


def _mlp(p1, p2, W1, b1, W2, b2):
    return pl.pallas_call(
        _mlp_body,
        out_shape=jax.ShapeDtypeStruct((B, 2), jnp.float32),
    )(p1, p2, W1, b1.reshape(1, -1), W2, b2.reshape(1, -1))


def kernel(data_x1, mask_x1, data_x2, mask_x2, word_embed, W1, b1, W2, b2):
    idx = jnp.stack([data_x1, data_x2]).astype(jnp.int32)
    idx = idx.reshape(F, NS, NCH, CR)
    maskf = jnp.stack([mask_x1, mask_x2]).astype(jnp.float32)
    maskf = jnp.pad(maskf, ((0, 0), (0, 0), (0, D - L)))
    maskf = maskf.reshape(F, NS, NB, D)
    pooled = _pool(idx, maskf, word_embed)
    return _mlp(pooled[0], pooled[1], W1, b1, W2, b2)
